# baseline (speedup 1.0000x reference)
import functools

import jax
import jax.numpy as jnp
from jax import lax
from jax.experimental import pallas as pl
from jax.experimental.pallas import tpu as pltpu
from jax.experimental.pallas import tpu_sc as plsc

N_USERS = 5000
N_ITEMS = 5000
N_NODES = N_USERS + N_ITEMS
N_EDGES = 320000
DIM = 128
N_HOPS = 3
BATCH = 1024
N_NEGS = 16
TEMPERATURE = 0.1
DECAY = 1e-4

NC = 2
NS = 16
NW = NC * NS

ECHUNK = 128
N_PAD = 10112
NZC = N_PAD // ECHUNK

CB = 80
NCHP = CB * NW
E_PAD = NCHP * ECHUNK
PASS = CB // 2
NPAIR = PASS // 2

GROW = 64
N_GROWS_P = 20480
GCH = N_GROWS_P // (NW * GROW)
N_GROWS = BATCH + BATCH * (1 + N_NEGS)


def _hop_body(table, col2, row2, val2, out,
              cidx2, ridx2, valv2, rows0, rows1, acc,
              gs0, gs1, ss0, ss1):
    cid = lax.axis_index("c")
    sid = lax.axis_index("s")
    wid = sid * NC + cid

    zf = jnp.zeros((16,), jnp.float32)

    def _zrow(i, c):
        for j in range(8):
            rows0[i, pl.ds(j * 16, 16)] = zf
        return c

    lax.fori_loop(0, ECHUNK, _zrow, 0)
    nzc = (NZC - sid + (NS - 1)) // NS

    def _zchunk(zi, c):
        r0 = (sid + zi * NS) * ECHUNK
        pltpu.async_copy(rows0, acc.at[pl.ds(r0, ECHUNK)], ss0)
        return c

    lax.fori_loop(0, nzc, _zchunk, 0)

    def _zdrain(zi, c):
        pltpu.make_async_copy(rows0, acc.at[pl.ds(0, ECHUNK)], ss0).wait()
        return c

    lax.fori_loop(0, nzc, _zdrain, 0)
    plsc.subcore_barrier()

    def _gstart(c, buf, sem):
        pltpu.async_copy(table.at[cidx2.at[c]], buf, sem)

    def _gwait(buf, sem):
        pltpu.make_async_copy(table.at[pl.ds(0, ECHUNK)], buf, sem).wait()

    def _sstart(c, buf, sem):
        pltpu.async_copy(buf, acc.at[ridx2.at[c]], sem, add=True)

    def _swait(buf, sem):
        pltpu.make_async_copy(buf, acc.at[pl.ds(0, ECHUNK)], sem).wait()

    def _scale(c, buf):
        def _g(g, cc):
            vv = valv2[c, pl.ds(g * 16, 16)]
            for r in range(16):
                vs = jnp.zeros((16,), jnp.float32) + vv[r]
                i = g * 16 + r
                for j in range(8):
                    s = pl.ds(j * 16, 16)
                    buf[i, s] = buf[i, s] * vs
            return cc

        lax.fori_loop(0, ECHUNK // 16, _g, 0)

    def _pair(i, c):
        c0 = 2 * i
        c1 = c0 + 1
        _gwait(rows0, gs0)
        _scale(c0, rows0)
        _sstart(c0, rows0, ss0)
        _gwait(rows1, gs1)
        _scale(c1, rows1)
        _sstart(c1, rows1, ss1)

        @pl.when(i + 1 < NPAIR)
        def _():
            _swait(rows0, ss0)
            _gstart(c0 + 2, rows0, gs0)
            _swait(rows1, ss1)
            _gstart(c1 + 2, rows1, gs1)

        return c

    for p in range(2):
        cstart = wid * CB + p * PASS
        pltpu.sync_copy(col2.at[pl.ds(cstart, PASS)], cidx2)
        pltpu.sync_copy(row2.at[pl.ds(cstart, PASS)], ridx2)
        pltpu.sync_copy(val2.at[pl.ds(cstart, PASS)], valv2)
        _gstart(0, rows0, gs0)
        _gstart(1, rows1, gs1)
        lax.fori_loop(0, NPAIR, _pair, 0)
        _swait(rows0, ss0)
        _swait(rows1, ss1)
    plsc.subcore_barrier()

    def _wchunk(zi, c):
        r0 = (sid + zi * NS) * ECHUNK
        pltpu.async_copy(acc.at[pl.ds(r0, ECHUNK)], out.at[cid, pl.ds(r0, ECHUNK)], ss0)
        return c

    lax.fori_loop(0, nzc, _wchunk, 0)

    def _wdrain(zi, c):
        pltpu.make_async_copy(acc.at[pl.ds(0, ECHUNK)], out.at[cid, pl.ds(0, ECHUNK)], ss0).wait()
        return c

    lax.fori_loop(0, nzc, _wdrain, 0)


def _make_hop():
    mesh = plsc.VectorSubcoreMesh(
        core_axis_name="c", subcore_axis_name="s", num_cores=NC, num_subcores=NS
    )
    return pl.kernel(
        _hop_body,
        out_type=jax.ShapeDtypeStruct((NC, N_PAD, DIM), jnp.float32),
        mesh=mesh,
        scratch_types=[
            pltpu.VMEM((PASS, ECHUNK), jnp.int32),
            pltpu.VMEM((PASS, ECHUNK), jnp.int32),
            pltpu.VMEM((PASS, ECHUNK), jnp.float32),
            pltpu.VMEM((ECHUNK, DIM), jnp.float32),
            pltpu.VMEM((ECHUNK, DIM), jnp.float32),
            pltpu.VMEM_SHARED((N_PAD, DIM), jnp.float32),
            pltpu.SemaphoreType.DMA,
            pltpu.SemaphoreType.DMA,
            pltpu.SemaphoreType.DMA,
            pltpu.SemaphoreType.DMA,
        ],
    )


def _batch_body(t0, t1, t2, t3, idx, out_sum, out_sq,
                idx_v, b00, b01, b02, b03, b10, b11, b12, b13,
                s0, s1, sq_v, gsem0, gsem1, osem):
    cid = lax.axis_index("c")
    sid = lax.axis_index("s")
    wid = sid * NC + cid
    zf = jnp.zeros((16,), jnp.float32)
    for j in range(8):
        sq_v[pl.ds(j * 16, 16)] = zf

    pltpu.sync_copy(idx.at[pl.ds(wid * (GCH * GROW), GCH * GROW)], idx_v)

    tabs = (t0, t1, t2, t3)

    def _gstart(k, bufs, sem):
        ix = idx_v.at[pl.ds(k * GROW, GROW)]
        for t, b in zip(tabs, bufs):
            pltpu.async_copy(t.at[ix], b, sem)

    def _gwait(bufs, sem):
        for b in bufs:
            pltpu.make_async_copy(t0.at[pl.ds(0, GROW)], b, sem).wait()

    def _rows(bufs, sbuf):
        def _row(i, cc):
            for j in range(8):
                s = pl.ds(j * 16, 16)
                x0 = bufs[0][i, s]
                x1 = bufs[1][i, s]
                x2 = bufs[2][i, s]
                x3 = bufs[3][i, s]
                sbuf[i, s] = (x0 + x1) + (x2 + x3)
                sq_v[s] = sq_v[s] + (x0 * x0 + x1 * x1) + (x2 * x2 + x3 * x3)
            return cc

        lax.fori_loop(0, GROW, _row, 0)

    bufs0 = (b00, b01, b02, b03)
    bufs1 = (b10, b11, b12, b13)
    base = wid * (GCH * GROW)

    _gstart(0, bufs0, gsem0)
    _gstart(1, bufs1, gsem1)
    for k in range(GCH):
        bufs = bufs0 if k % 2 == 0 else bufs1
        sem = gsem0 if k % 2 == 0 else gsem1
        sbuf = s0 if k % 2 == 0 else s1
        _gwait(bufs, sem)
        if k >= 2:
            pltpu.make_async_copy(sbuf, out_sum.at[pl.ds(0, GROW)], osem).wait()
        _rows(bufs, sbuf)
        if k + 2 < GCH:
            _gstart(k + 2, bufs, sem)
        pltpu.async_copy(sbuf, out_sum.at[pl.ds(base + k * GROW, GROW)], osem)
    for k in (GCH - 2, GCH - 1):
        sbuf = s0 if k % 2 == 0 else s1
        pltpu.make_async_copy(sbuf, out_sum.at[pl.ds(0, GROW)], osem).wait()

    pltpu.sync_copy(sq_v, out_sq.at[pl.ds(wid * 128, 128)])


def _make_batch_gather():
    mesh = plsc.VectorSubcoreMesh(
        core_axis_name="c", subcore_axis_name="s", num_cores=NC, num_subcores=NS
    )
    return pl.kernel(
        _batch_body,
        out_type=(
            jax.ShapeDtypeStruct((N_GROWS_P, DIM), jnp.float32),
            jax.ShapeDtypeStruct((NW * 128,), jnp.float32),
        ),
        mesh=mesh,
        scratch_types=[
            pltpu.VMEM((GCH * GROW,), jnp.int32),
            pltpu.VMEM((GROW, DIM), jnp.float32),
            pltpu.VMEM((GROW, DIM), jnp.float32),
            pltpu.VMEM((GROW, DIM), jnp.float32),
            pltpu.VMEM((GROW, DIM), jnp.float32),
            pltpu.VMEM((GROW, DIM), jnp.float32),
            pltpu.VMEM((GROW, DIM), jnp.float32),
            pltpu.VMEM((GROW, DIM), jnp.float32),
            pltpu.VMEM((GROW, DIM), jnp.float32),
            pltpu.VMEM((GROW, DIM), jnp.float32),
            pltpu.VMEM((GROW, DIM), jnp.float32),
            pltpu.VMEM((128,), jnp.float32),
            pltpu.SemaphoreType.DMA,
            pltpu.SemaphoreType.DMA,
            pltpu.SemaphoreType.DMA,
        ],
    )


def _comb_body(p_ref, o_ref):
    o_ref[...] = p_ref[0] + p_ref[1]


def _combine(partial):
    return pl.pallas_call(
        _comb_body,
        grid=(NZC,),
        in_specs=[pl.BlockSpec((2, ECHUNK, DIM), lambda i: (0, i, 0))],
        out_specs=pl.BlockSpec((ECHUNK, DIM), lambda i: (i, 0)),
        out_shape=jax.ShapeDtypeStruct((N_PAD, DIM), jnp.float32),
    )(partial)


def _loss_body(u_ref, it_ref, sq_ref, o_ref):
    i = pl.program_id(0)
    u = u_ref[...]
    un = jnp.maximum(jnp.sqrt(jnp.sum(u * u, axis=1, keepdims=True)), 1e-12)
    uh = u / un
    it = it_ref[...]
    inorm = jnp.maximum(jnp.sqrt(jnp.sum(it * it, axis=2, keepdims=True)), 1e-12)
    y = jnp.sum((it / inorm) * uh[:, None, :], axis=2)
    logits = y / TEMPERATURE
    m = jnp.max(logits, axis=1, keepdims=True)
    lse = m + jnp.log(jnp.sum(jnp.exp(logits - m), axis=1, keepdims=True))
    part = jnp.sum(lse - logits[:, 0:1]) / BATCH

    @pl.when(i == 0)
    def _():
        o_ref[...] = jnp.full(
            (8, 128), DECAY * jnp.sum(sq_ref[...]) / (2.0 * BATCH), jnp.float32
        )

    o_ref[...] = o_ref[...] + part


def _loss(u_sum, items, sq):
    bb = 128
    return pl.pallas_call(
        _loss_body,
        grid=(BATCH // bb,),
        in_specs=[
            pl.BlockSpec((bb, DIM), lambda i: (i, 0)),
            pl.BlockSpec((bb, 1 + N_NEGS, DIM), lambda i: (i, 0, 0)),
            pl.BlockSpec((NW, 128), lambda i: (0, 0)),
        ],
        out_specs=pl.BlockSpec((8, 128), lambda i: (0, 0)),
        out_shape=jax.ShapeDtypeStruct((8, 128), jnp.float32),
    )(u_sum, items, sq)


def kernel(user_embed, item_embed, edge_index, edge_val, users, pos_items, neg_items):
    row = edge_index[0].astype(jnp.int32)
    col = edge_index[1].astype(jnp.int32)
    t0 = jnp.concatenate(
        [user_embed, item_embed,
         jnp.zeros((N_PAD - N_NODES, DIM), jnp.float32)], axis=0
    )

    npad = E_PAD - N_EDGES
    pad_idx = (jnp.arange(npad, dtype=jnp.int32) * 61) % N_NODES
    col2 = jnp.concatenate([col, pad_idx]).reshape(NCHP, ECHUNK)
    row2 = jnp.concatenate([row, pad_idx]).reshape(NCHP, ECHUNK)
    val2 = jnp.concatenate(
        [edge_val, jnp.zeros((npad,), jnp.float32)]
    ).reshape(NCHP, ECHUNK)

    hop = _make_hop()
    tables = [t0]
    t = t0
    for _ in range(N_HOPS):
        partial = hop(t, col2, row2, val2)
        t = _combine(partial)
        tables.append(t)

    item_idx = jnp.concatenate(
        [pos_items[:, None].astype(jnp.int32), neg_items.astype(jnp.int32)], axis=1
    )
    idx_all = jnp.concatenate(
        [users.astype(jnp.int32), (item_idx + N_USERS).reshape(-1),
         jnp.full((N_GROWS_P - N_GROWS,), N_NODES, jnp.int32)]
    )
    sums, sq = _make_batch_gather()(tables[0], tables[1], tables[2], tables[3], idx_all)
    u_sum = sums[:BATCH]
    items = sums[BATCH:N_GROWS].reshape(BATCH, 1 + N_NEGS, DIM)
    out = _loss(u_sum, items, sq.reshape(NW, 128))
    return out[0, 0]


# speedup vs baseline: 1.0711x; 1.0711x over previous
import functools

import jax
import jax.numpy as jnp
from jax import lax
from jax.experimental import pallas as pl
from jax.experimental.pallas import tpu as pltpu
from jax.experimental.pallas import tpu_sc as plsc

N_USERS = 5000
N_ITEMS = 5000
N_NODES = N_USERS + N_ITEMS
N_EDGES = 320000
DIM = 128
N_HOPS = 3
BATCH = 1024
N_NEGS = 16
TEMPERATURE = 0.1
DECAY = 1e-4

NC = 2
NS = 16
NW = NC * NS

ECHUNK = 128
N_PAD = 10112
NZC = N_PAD // ECHUNK

CB = 80
NCHP = CB * NW
E_PAD = NCHP * ECHUNK
PASS = CB // 2
NPAIR = PASS // 2

N_GROWS = BATCH + BATCH * (1 + N_NEGS)
NGCH = N_GROWS // ECHUNK


def _hop_body(table, col2, row2, val2, out,
              cidx2, ridx2, valv2, rows0, rows1, acc,
              gs0, gs1, ss0, ss1):
    cid = lax.axis_index("c")
    sid = lax.axis_index("s")
    wid = sid * NC + cid

    zf = jnp.zeros((16,), jnp.float32)

    def _zrow(i, c):
        for j in range(8):
            rows0[i, pl.ds(j * 16, 16)] = zf
        return c

    lax.fori_loop(0, ECHUNK, _zrow, 0)
    nzc = (NZC - sid + (NS - 1)) // NS

    def _zchunk(zi, c):
        r0 = (sid + zi * NS) * ECHUNK
        pltpu.async_copy(rows0, acc.at[pl.ds(r0, ECHUNK)], ss0)
        return c

    lax.fori_loop(0, nzc, _zchunk, 0)

    def _zdrain(zi, c):
        pltpu.make_async_copy(rows0, acc.at[pl.ds(0, ECHUNK)], ss0).wait()
        return c

    lax.fori_loop(0, nzc, _zdrain, 0)
    plsc.subcore_barrier()

    def _gstart(c, buf, sem):
        pltpu.async_copy(table.at[cidx2.at[c]], buf, sem)

    def _gwait(buf, sem):
        pltpu.make_async_copy(table.at[pl.ds(0, ECHUNK)], buf, sem).wait()

    def _sstart(c, buf, sem):
        pltpu.async_copy(buf, acc.at[ridx2.at[c]], sem, add=True)

    def _swait(buf, sem):
        pltpu.make_async_copy(buf, acc.at[pl.ds(0, ECHUNK)], sem).wait()

    def _scale(c, buf):
        def _g(g, cc):
            vv = valv2[c, pl.ds(g * 16, 16)]
            for r in range(16):
                vs = jnp.zeros((16,), jnp.float32) + vv[r]
                i = g * 16 + r
                for j in range(8):
                    s = pl.ds(j * 16, 16)
                    buf[i, s] = buf[i, s] * vs
            return cc

        lax.fori_loop(0, ECHUNK // 16, _g, 0)

    def _pair(i, c):
        c0 = 2 * i
        c1 = c0 + 1
        _gwait(rows0, gs0)
        _scale(c0, rows0)
        _sstart(c0, rows0, ss0)
        _gwait(rows1, gs1)
        _scale(c1, rows1)
        _sstart(c1, rows1, ss1)

        @pl.when(i + 1 < NPAIR)
        def _():
            _swait(rows0, ss0)
            _gstart(c0 + 2, rows0, gs0)
            _swait(rows1, ss1)
            _gstart(c1 + 2, rows1, gs1)

        return c

    for p in range(2):
        cstart = wid * CB + p * PASS
        pltpu.sync_copy(col2.at[pl.ds(cstart, PASS)], cidx2)
        pltpu.sync_copy(row2.at[pl.ds(cstart, PASS)], ridx2)
        pltpu.sync_copy(val2.at[pl.ds(cstart, PASS)], valv2)
        _gstart(0, rows0, gs0)
        _gstart(1, rows1, gs1)
        lax.fori_loop(0, NPAIR, _pair, 0)
        _swait(rows0, ss0)
        _swait(rows1, ss1)
    plsc.subcore_barrier()

    def _wchunk(zi, c):
        r0 = (sid + zi * NS) * ECHUNK
        pltpu.async_copy(acc.at[pl.ds(r0, ECHUNK)], out.at[cid, pl.ds(r0, ECHUNK)], ss0)
        return c

    lax.fori_loop(0, nzc, _wchunk, 0)

    def _wdrain(zi, c):
        pltpu.make_async_copy(acc.at[pl.ds(0, ECHUNK)], out.at[cid, pl.ds(0, ECHUNK)], ss0).wait()
        return c

    lax.fori_loop(0, nzc, _wdrain, 0)


def _make_hop():
    mesh = plsc.VectorSubcoreMesh(
        core_axis_name="c", subcore_axis_name="s", num_cores=NC, num_subcores=NS
    )
    return pl.kernel(
        _hop_body,
        out_type=jax.ShapeDtypeStruct((NC, N_PAD, DIM), jnp.float32),
        mesh=mesh,
        scratch_types=[
            pltpu.VMEM((PASS, ECHUNK), jnp.int32),
            pltpu.VMEM((PASS, ECHUNK), jnp.int32),
            pltpu.VMEM((PASS, ECHUNK), jnp.float32),
            pltpu.VMEM((ECHUNK, DIM), jnp.float32),
            pltpu.VMEM((ECHUNK, DIM), jnp.float32),
            pltpu.VMEM_SHARED((N_PAD, DIM), jnp.float32),
            pltpu.SemaphoreType.DMA,
            pltpu.SemaphoreType.DMA,
            pltpu.SemaphoreType.DMA,
            pltpu.SemaphoreType.DMA,
        ],
    )


def _batch_body(t0, t1, t2, t3, idx, out_sum, out_sq,
                idx_v, b0, b1, b2, b3, sq_v, sem):
    cid = lax.axis_index("c")
    sid = lax.axis_index("s")
    wid = sid * NC + cid
    zf = jnp.zeros((16,), jnp.float32)
    for j in range(8):
        sq_v[pl.ds(j * 16, 16)] = zf
    nch = (NGCH - wid + (NW - 1)) // NW

    def _chunk(ci, c):
        base = (wid + ci * NW) * ECHUNK
        pltpu.sync_copy(idx.at[pl.ds(base, ECHUNK)], idx_v)
        d0 = pltpu.async_copy(t0.at[idx_v], b0, sem)
        d1 = pltpu.async_copy(t1.at[idx_v], b1, sem)
        d2 = pltpu.async_copy(t2.at[idx_v], b2, sem)
        d3 = pltpu.async_copy(t3.at[idx_v], b3, sem)
        d0.wait()
        d1.wait()
        d2.wait()
        d3.wait()

        def _row(i, cc):
            for j in range(8):
                s = pl.ds(j * 16, 16)
                x0 = b0[i, s]
                x1 = b1[i, s]
                x2 = b2[i, s]
                x3 = b3[i, s]
                b0[i, s] = (x0 + x1) + (x2 + x3)
                sq_v[s] = sq_v[s] + (x0 * x0 + x1 * x1) + (x2 * x2 + x3 * x3)
            return cc

        lax.fori_loop(0, ECHUNK, _row, 0)
        pltpu.sync_copy(b0, out_sum.at[pl.ds(base, ECHUNK)])
        return c

    lax.fori_loop(0, nch, _chunk, 0)
    pltpu.sync_copy(sq_v, out_sq.at[pl.ds(wid * 128, 128)])


def _make_batch_gather():
    mesh = plsc.VectorSubcoreMesh(
        core_axis_name="c", subcore_axis_name="s", num_cores=NC, num_subcores=NS
    )
    return pl.kernel(
        _batch_body,
        out_type=(
            jax.ShapeDtypeStruct((N_GROWS, DIM), jnp.float32),
            jax.ShapeDtypeStruct((NW * 128,), jnp.float32),
        ),
        mesh=mesh,
        scratch_types=[
            pltpu.VMEM((ECHUNK,), jnp.int32),
            pltpu.VMEM((ECHUNK, DIM), jnp.float32),
            pltpu.VMEM((ECHUNK, DIM), jnp.float32),
            pltpu.VMEM((ECHUNK, DIM), jnp.float32),
            pltpu.VMEM((ECHUNK, DIM), jnp.float32),
            pltpu.VMEM((128,), jnp.float32),
            pltpu.SemaphoreType.DMA,
        ],
    )


def _comb_body(p_ref, o_ref):
    o_ref[...] = p_ref[0] + p_ref[1]


def _combine(partial):
    return pl.pallas_call(
        _comb_body,
        grid=(NZC,),
        in_specs=[pl.BlockSpec((2, ECHUNK, DIM), lambda i: (0, i, 0))],
        out_specs=pl.BlockSpec((ECHUNK, DIM), lambda i: (i, 0)),
        out_shape=jax.ShapeDtypeStruct((N_PAD, DIM), jnp.float32),
    )(partial)


def _loss_body(u_ref, it_ref, sq_ref, o_ref):
    i = pl.program_id(0)
    u = u_ref[...]
    un = jnp.maximum(jnp.sqrt(jnp.sum(u * u, axis=1, keepdims=True)), 1e-12)
    uh = u / un
    it = it_ref[...]
    inorm = jnp.maximum(jnp.sqrt(jnp.sum(it * it, axis=2, keepdims=True)), 1e-12)
    y = jnp.sum((it / inorm) * uh[:, None, :], axis=2)
    logits = y / TEMPERATURE
    m = jnp.max(logits, axis=1, keepdims=True)
    lse = m + jnp.log(jnp.sum(jnp.exp(logits - m), axis=1, keepdims=True))
    part = jnp.sum(lse - logits[:, 0:1]) / BATCH

    @pl.when(i == 0)
    def _():
        o_ref[...] = jnp.full(
            (8, 128), DECAY * jnp.sum(sq_ref[...]) / (2.0 * BATCH), jnp.float32
        )

    o_ref[...] = o_ref[...] + part


def _loss(u_sum, items, sq):
    bb = 128
    return pl.pallas_call(
        _loss_body,
        grid=(BATCH // bb,),
        in_specs=[
            pl.BlockSpec((bb, DIM), lambda i: (i, 0)),
            pl.BlockSpec((bb, 1 + N_NEGS, DIM), lambda i: (i, 0, 0)),
            pl.BlockSpec((NW, 128), lambda i: (0, 0)),
        ],
        out_specs=pl.BlockSpec((8, 128), lambda i: (0, 0)),
        out_shape=jax.ShapeDtypeStruct((8, 128), jnp.float32),
    )(u_sum, items, sq)


def kernel(user_embed, item_embed, edge_index, edge_val, users, pos_items, neg_items):
    row = edge_index[0].astype(jnp.int32)
    col = edge_index[1].astype(jnp.int32)
    t0 = jnp.concatenate(
        [user_embed, item_embed,
         jnp.zeros((N_PAD - N_NODES, DIM), jnp.float32)], axis=0
    )

    npad = E_PAD - N_EDGES
    pad_idx = (jnp.arange(npad, dtype=jnp.int32) * 61) % N_NODES
    col2 = jnp.concatenate([col, pad_idx]).reshape(NCHP, ECHUNK)
    row2 = jnp.concatenate([row, pad_idx]).reshape(NCHP, ECHUNK)
    val2 = jnp.concatenate(
        [edge_val, jnp.zeros((npad,), jnp.float32)]
    ).reshape(NCHP, ECHUNK)

    hop = _make_hop()
    tables = [t0]
    t = t0
    for _ in range(N_HOPS):
        partial = hop(t, col2, row2, val2)
        t = _combine(partial)
        tables.append(t)

    item_idx = jnp.concatenate(
        [pos_items[:, None].astype(jnp.int32), neg_items.astype(jnp.int32)], axis=1
    )
    idx_all = jnp.concatenate(
        [users.astype(jnp.int32), (item_idx + N_USERS).reshape(-1)]
    )
    sums, sq = _make_batch_gather()(tables[0], tables[1], tables[2], tables[3], idx_all)
    u_sum = sums[:BATCH]
    items = sums[BATCH:].reshape(BATCH, 1 + N_NEGS, DIM)
    out = _loss(u_sum, items, sq.reshape(NW, 128))
    return out[0, 0]


# speedup vs baseline: 1.1367x; 1.0613x over previous
import functools

import jax
import jax.numpy as jnp
from jax import lax
from jax.experimental import pallas as pl
from jax.experimental.pallas import tpu as pltpu
from jax.experimental.pallas import tpu_sc as plsc

N_USERS = 5000
N_ITEMS = 5000
N_NODES = N_USERS + N_ITEMS
N_EDGES = 320000
DIM = 128
N_HOPS = 3
BATCH = 1024
N_NEGS = 16
TEMPERATURE = 0.1
DECAY = 1e-4

NC = 2
NS = 16
NW = NC * NS

ECHUNK = 128
N_PAD = 10112
NZC = N_PAD // ECHUNK

CB = 80
NCHP = CB * NW
E_PAD = NCHP * ECHUNK
PASS = CB // 2
NPAIR = PASS // 2

N_GROWS = BATCH + BATCH * (1 + N_NEGS)
NGCH = N_GROWS // ECHUNK


def _hop_body(table, col2, row2, val2, out_a, out_b,
              cidx2, ridx2, valv2, rows0, rows1, acc,
              gs0, gs1, ss0, ss1):
    cid = lax.axis_index("c")
    sid = lax.axis_index("s")
    wid = sid * NC + cid

    zf = jnp.zeros((16,), jnp.float32)

    def _zrow(i, c):
        for j in range(8):
            rows0[i, pl.ds(j * 16, 16)] = zf
        return c

    lax.fori_loop(0, ECHUNK, _zrow, 0)
    nzc = (NZC - sid + (NS - 1)) // NS

    def _zchunk(zi, c):
        r0 = (sid + zi * NS) * ECHUNK
        pltpu.async_copy(rows0, acc.at[pl.ds(r0, ECHUNK)], ss0)
        return c

    lax.fori_loop(0, nzc, _zchunk, 0)

    def _zdrain(zi, c):
        pltpu.make_async_copy(rows0, acc.at[pl.ds(0, ECHUNK)], ss0).wait()
        return c

    lax.fori_loop(0, nzc, _zdrain, 0)
    plsc.subcore_barrier()

    def _gstart(c, buf, sem):
        pltpu.async_copy(table.at[cidx2.at[c]], buf, sem)

    def _gwait(buf, sem):
        pltpu.make_async_copy(table.at[pl.ds(0, ECHUNK)], buf, sem).wait()

    def _sstart(c, buf, sem):
        pltpu.async_copy(buf, acc.at[ridx2.at[c]], sem, add=True)

    def _swait(buf, sem):
        pltpu.make_async_copy(buf, acc.at[pl.ds(0, ECHUNK)], sem).wait()

    def _scale(c, buf):
        def _g(g, cc):
            vv = valv2[c, pl.ds(g * 16, 16)]
            for r in range(16):
                vs = jnp.zeros((16,), jnp.float32) + vv[r]
                i = g * 16 + r
                for j in range(8):
                    s = pl.ds(j * 16, 16)
                    buf[i, s] = buf[i, s] * vs
            return cc

        lax.fori_loop(0, ECHUNK // 16, _g, 0)

    def _pair(i, c):
        c0 = 2 * i
        c1 = c0 + 1
        _gwait(rows0, gs0)
        _scale(c0, rows0)
        _sstart(c0, rows0, ss0)
        _gwait(rows1, gs1)
        _scale(c1, rows1)
        _sstart(c1, rows1, ss1)

        @pl.when(i + 1 < NPAIR)
        def _():
            _swait(rows0, ss0)
            _gstart(c0 + 2, rows0, gs0)
            _swait(rows1, ss1)
            _gstart(c1 + 2, rows1, gs1)

        return c

    for p in range(2):
        cstart = wid * CB + p * PASS
        pltpu.sync_copy(col2.at[pl.ds(cstart, PASS)], cidx2)
        pltpu.sync_copy(row2.at[pl.ds(cstart, PASS)], ridx2)
        pltpu.sync_copy(val2.at[pl.ds(cstart, PASS)], valv2)
        _gstart(0, rows0, gs0)
        _gstart(1, rows1, gs1)
        lax.fori_loop(0, NPAIR, _pair, 0)
        _swait(rows0, ss0)
        _swait(rows1, ss1)
    plsc.subcore_barrier()

    @pl.when(cid == 0)
    def _():
        def _wchunk(zi, c):
            r0 = (sid + zi * NS) * ECHUNK
            pltpu.async_copy(acc.at[pl.ds(r0, ECHUNK)], out_a.at[pl.ds(r0, ECHUNK)], ss0)
            return c

        lax.fori_loop(0, nzc, _wchunk, 0)

    @pl.when(cid == 1)
    def _():
        def _wchunk(zi, c):
            r0 = (sid + zi * NS) * ECHUNK
            pltpu.async_copy(acc.at[pl.ds(r0, ECHUNK)], out_b.at[pl.ds(r0, ECHUNK)], ss0)
            return c

        lax.fori_loop(0, nzc, _wchunk, 0)

    def _wdrain(zi, c):
        pltpu.make_async_copy(acc.at[pl.ds(0, ECHUNK)], out_a.at[pl.ds(0, ECHUNK)], ss0).wait()
        return c

    lax.fori_loop(0, nzc, _wdrain, 0)


def _make_hop():
    mesh = plsc.VectorSubcoreMesh(
        core_axis_name="c", subcore_axis_name="s", num_cores=NC, num_subcores=NS
    )
    return pl.kernel(
        _hop_body,
        out_type=(
            jax.ShapeDtypeStruct((N_PAD, DIM), jnp.float32),
            jax.ShapeDtypeStruct((N_PAD, DIM), jnp.float32),
        ),
        mesh=mesh,
        scratch_types=[
            pltpu.VMEM((PASS, ECHUNK), jnp.int32),
            pltpu.VMEM((PASS, ECHUNK), jnp.int32),
            pltpu.VMEM((PASS, ECHUNK), jnp.float32),
            pltpu.VMEM((ECHUNK, DIM), jnp.float32),
            pltpu.VMEM((ECHUNK, DIM), jnp.float32),
            pltpu.VMEM_SHARED((N_PAD, DIM), jnp.float32),
            pltpu.SemaphoreType.DMA,
            pltpu.SemaphoreType.DMA,
            pltpu.SemaphoreType.DMA,
            pltpu.SemaphoreType.DMA,
        ],
    )


def _batch_body(t0, t1, t2, p3a, p3b, idx, out_u, out_i, out_sq,
                idx_v, b0, b1, b2, b3, b4, sq_v, sem):
    cid = lax.axis_index("c")
    sid = lax.axis_index("s")
    wid = sid * NC + cid
    zf = jnp.zeros((16,), jnp.float32)
    for j in range(8):
        sq_v[pl.ds(j * 16, 16)] = zf
    nch = (NGCH - wid + (NW - 1)) // NW
    UCH = BATCH // ECHUNK

    def _chunk(ci, c):
        ch = wid + ci * NW
        base = ch * ECHUNK
        pltpu.sync_copy(idx.at[pl.ds(base, ECHUNK)], idx_v)
        d0 = pltpu.async_copy(t0.at[idx_v], b0, sem)
        d1 = pltpu.async_copy(t1.at[idx_v], b1, sem)
        d2 = pltpu.async_copy(t2.at[idx_v], b2, sem)
        d3 = pltpu.async_copy(p3a.at[idx_v], b3, sem)
        d4 = pltpu.async_copy(p3b.at[idx_v], b4, sem)
        d0.wait()
        d1.wait()
        d2.wait()
        d3.wait()
        d4.wait()

        def _row(i, cc):
            for j in range(8):
                s = pl.ds(j * 16, 16)
                x0 = b0[i, s]
                x1 = b1[i, s]
                x2 = b2[i, s]
                x3 = b3[i, s] + b4[i, s]
                b0[i, s] = (x0 + x1) + (x2 + x3)
                sq_v[s] = sq_v[s] + (x0 * x0 + x1 * x1) + (x2 * x2 + x3 * x3)
            return cc

        lax.fori_loop(0, ECHUNK, _row, 0)

        @pl.when(ch < UCH)
        def _():
            pltpu.sync_copy(b0, out_u.at[pl.ds(base, ECHUNK)])

        @pl.when(ch >= UCH)
        def _():
            pltpu.sync_copy(b0, out_i.at[pl.ds(base - BATCH, ECHUNK)])

        return c

    lax.fori_loop(0, nch, _chunk, 0)
    pltpu.sync_copy(sq_v, out_sq.at[pl.ds(wid * 128, 128)])


def _make_batch_gather():
    mesh = plsc.VectorSubcoreMesh(
        core_axis_name="c", subcore_axis_name="s", num_cores=NC, num_subcores=NS
    )
    return pl.kernel(
        _batch_body,
        out_type=(
            jax.ShapeDtypeStruct((BATCH, DIM), jnp.float32),
            jax.ShapeDtypeStruct((N_GROWS - BATCH, DIM), jnp.float32),
            jax.ShapeDtypeStruct((NW * 128,), jnp.float32),
        ),
        mesh=mesh,
        scratch_types=[
            pltpu.VMEM((ECHUNK,), jnp.int32),
            pltpu.VMEM((ECHUNK, DIM), jnp.float32),
            pltpu.VMEM((ECHUNK, DIM), jnp.float32),
            pltpu.VMEM((ECHUNK, DIM), jnp.float32),
            pltpu.VMEM((ECHUNK, DIM), jnp.float32),
            pltpu.VMEM((ECHUNK, DIM), jnp.float32),
            pltpu.VMEM((128,), jnp.float32),
            pltpu.SemaphoreType.DMA,
        ],
    )


def _comb_body(a_ref, b_ref, o_ref):
    o_ref[...] = a_ref[...] + b_ref[...]


def _combine(pa, pb):
    return pl.pallas_call(
        _comb_body,
        grid=(NZC,),
        in_specs=[pl.BlockSpec((ECHUNK, DIM), lambda i: (i, 0))] * 2,
        out_specs=pl.BlockSpec((ECHUNK, DIM), lambda i: (i, 0)),
        out_shape=jax.ShapeDtypeStruct((N_PAD, DIM), jnp.float32),
    )(pa, pb)


def _loss_body(u_ref, it_ref, sq_ref, o_ref):
    i = pl.program_id(0)
    u = u_ref[...]
    un = jnp.maximum(jnp.sqrt(jnp.sum(u * u, axis=1, keepdims=True)), 1e-12)
    uh = u / un
    it = it_ref[...]
    inorm = jnp.maximum(jnp.sqrt(jnp.sum(it * it, axis=2, keepdims=True)), 1e-12)
    y = jnp.sum((it / inorm) * uh[:, None, :], axis=2)
    logits = y / TEMPERATURE
    m = jnp.max(logits, axis=1, keepdims=True)
    lse = m + jnp.log(jnp.sum(jnp.exp(logits - m), axis=1, keepdims=True))
    part = jnp.sum(lse - logits[:, 0:1]) / BATCH

    @pl.when(i == 0)
    def _():
        o_ref[...] = jnp.full(
            (8, 128), DECAY * jnp.sum(sq_ref[...]) / (2.0 * BATCH), jnp.float32
        )

    o_ref[...] = o_ref[...] + part


def _loss(u_sum, items, sq):
    bb = 128
    return pl.pallas_call(
        _loss_body,
        grid=(BATCH // bb,),
        in_specs=[
            pl.BlockSpec((bb, DIM), lambda i: (i, 0)),
            pl.BlockSpec((bb, 1 + N_NEGS, DIM), lambda i: (i, 0, 0)),
            pl.BlockSpec((NW, 128), lambda i: (0, 0)),
        ],
        out_specs=pl.BlockSpec((8, 128), lambda i: (0, 0)),
        out_shape=jax.ShapeDtypeStruct((8, 128), jnp.float32),
    )(u_sum, items, sq)


def kernel(user_embed, item_embed, edge_index, edge_val, users, pos_items, neg_items):
    row = edge_index[0].astype(jnp.int32)
    col = edge_index[1].astype(jnp.int32)
    t0 = jnp.concatenate(
        [user_embed, item_embed,
         jnp.zeros((N_PAD - N_NODES, DIM), jnp.float32)], axis=0
    )

    npad = E_PAD - N_EDGES
    pad_idx = (jnp.arange(npad, dtype=jnp.int32) * 61) % N_NODES
    col2 = jnp.concatenate([col, pad_idx]).reshape(NCHP, ECHUNK)
    row2 = jnp.concatenate([row, pad_idx]).reshape(NCHP, ECHUNK)
    val2 = jnp.concatenate(
        [edge_val, jnp.zeros((npad,), jnp.float32)]
    ).reshape(NCHP, ECHUNK)

    hop = _make_hop()
    t1 = _combine(*hop(t0, col2, row2, val2))
    t2 = _combine(*hop(t1, col2, row2, val2))
    p3a, p3b = hop(t2, col2, row2, val2)

    item_idx = jnp.concatenate(
        [pos_items[:, None].astype(jnp.int32), neg_items.astype(jnp.int32)], axis=1
    )
    idx_all = jnp.concatenate(
        [users.astype(jnp.int32), (item_idx + N_USERS).reshape(-1)]
    )
    u_sum, item_sum, sq = _make_batch_gather()(t0, t1, t2, p3a, p3b, idx_all)
    items = item_sum.reshape(BATCH, 1 + N_NEGS, DIM)
    out = _loss(u_sum, items, sq.reshape(NW, 128))
    return out[0, 0]


# speedup vs baseline: 1.1375x; 1.0007x over previous
import jax
import jax.numpy as jnp
from jax import lax
from jax.experimental import pallas as pl
from jax.experimental.pallas import tpu as pltpu
from jax.experimental.pallas import tpu_sc as plsc

N_USERS = 5000
N_ITEMS = 5000
N_NODES = N_USERS + N_ITEMS
N_EDGES = 320000
DIM = 128
N_HOPS = 3
BATCH = 1024
N_NEGS = 16
TEMPERATURE = 0.1
DECAY = 1e-4

NC = 2
NS = 16
NW = NC * NS

ECHUNK = 128
N_PAD = 10112
NZC = N_PAD // ECHUNK

CB = 80
NCHP = CB * NW
E_PAD = NCHP * ECHUNK
PASS = CB // 2
NPAIR = PASS // 2

N_GROWS = BATCH + BATCH * (1 + N_NEGS)
NGCH = N_GROWS // ECHUNK


def _hop_body(table, col2, row2, val2, out_a, out_b,
              cidx2, ridx2, valv2, rows0, rows1, acc,
              gs0, gs1, ss0, ss1):
    cid = lax.axis_index("c")
    sid = lax.axis_index("s")
    wid = sid * NC + cid

    zf = jnp.zeros((16,), jnp.float32)

    def _zrow(i, c):
        for j in range(8):
            rows0[i, pl.ds(j * 16, 16)] = zf
        return c

    lax.fori_loop(0, ECHUNK, _zrow, 0)
    nzc = (NZC - sid + (NS - 1)) // NS

    def _zchunk(zi, c):
        r0 = (sid + zi * NS) * ECHUNK
        pltpu.async_copy(rows0, acc.at[pl.ds(r0, ECHUNK)], ss0)
        return c

    lax.fori_loop(0, nzc, _zchunk, 0)

    def _zdrain(zi, c):
        pltpu.make_async_copy(rows0, acc.at[pl.ds(0, ECHUNK)], ss0).wait()
        return c

    lax.fori_loop(0, nzc, _zdrain, 0)
    plsc.subcore_barrier()

    def _gstart(c, buf, sem):
        pltpu.async_copy(table.at[cidx2.at[c]], buf, sem)

    def _gwait(buf, sem):
        pltpu.make_async_copy(table.at[pl.ds(0, ECHUNK)], buf, sem).wait()

    def _sstart(c, buf, sem):
        pltpu.async_copy(buf, acc.at[ridx2.at[c]], sem, add=True)

    def _swait(buf, sem):
        pltpu.make_async_copy(buf, acc.at[pl.ds(0, ECHUNK)], sem).wait()

    def _scale(c, buf):
        def _g(g, cc):
            vv = valv2[c, pl.ds(g * 16, 16)]
            for r in range(16):
                vs = jnp.zeros((16,), jnp.float32) + vv[r]
                i = g * 16 + r
                for j in range(8):
                    s = pl.ds(j * 16, 16)
                    buf[i, s] = buf[i, s] * vs
            return cc

        lax.fori_loop(0, ECHUNK // 16, _g, 0)

    def _pair(i, c):
        c0 = 2 * i
        c1 = c0 + 1
        _gwait(rows0, gs0)
        _scale(c0, rows0)
        _sstart(c0, rows0, ss0)
        _gwait(rows1, gs1)
        _scale(c1, rows1)
        _sstart(c1, rows1, ss1)

        @pl.when(i + 1 < NPAIR)
        def _():
            _swait(rows0, ss0)
            _gstart(c0 + 2, rows0, gs0)
            _swait(rows1, ss1)
            _gstart(c1 + 2, rows1, gs1)

        return c

    for p in range(2):
        cstart = wid * CB + p * PASS
        pltpu.sync_copy(col2.at[pl.ds(cstart, PASS)], cidx2)
        pltpu.sync_copy(row2.at[pl.ds(cstart, PASS)], ridx2)
        pltpu.sync_copy(val2.at[pl.ds(cstart, PASS)], valv2)
        _gstart(0, rows0, gs0)
        _gstart(1, rows1, gs1)
        lax.fori_loop(0, NPAIR, _pair, 0)
        _swait(rows0, ss0)
        _swait(rows1, ss1)
    plsc.subcore_barrier()

    @pl.when(cid == 0)
    def _():
        def _wchunk(zi, c):
            r0 = (sid + zi * NS) * ECHUNK
            pltpu.async_copy(acc.at[pl.ds(r0, ECHUNK)], out_a.at[pl.ds(r0, ECHUNK)], ss0)
            return c

        lax.fori_loop(0, nzc, _wchunk, 0)

    @pl.when(cid == 1)
    def _():
        def _wchunk(zi, c):
            r0 = (sid + zi * NS) * ECHUNK
            pltpu.async_copy(acc.at[pl.ds(r0, ECHUNK)], out_b.at[pl.ds(r0, ECHUNK)], ss0)
            return c

        lax.fori_loop(0, nzc, _wchunk, 0)

    def _wdrain(zi, c):
        pltpu.make_async_copy(acc.at[pl.ds(0, ECHUNK)], out_a.at[pl.ds(0, ECHUNK)], ss0).wait()
        return c

    lax.fori_loop(0, nzc, _wdrain, 0)


def _make_hop():
    mesh = plsc.VectorSubcoreMesh(
        core_axis_name="c", subcore_axis_name="s", num_cores=NC, num_subcores=NS
    )
    return pl.kernel(
        _hop_body,
        out_type=(
            jax.ShapeDtypeStruct((N_PAD, DIM), jnp.float32),
            jax.ShapeDtypeStruct((N_PAD, DIM), jnp.float32),
        ),
        mesh=mesh,
        scratch_types=[
            pltpu.VMEM((PASS, ECHUNK), jnp.int32),
            pltpu.VMEM((PASS, ECHUNK), jnp.int32),
            pltpu.VMEM((PASS, ECHUNK), jnp.float32),
            pltpu.VMEM((ECHUNK, DIM), jnp.float32),
            pltpu.VMEM((ECHUNK, DIM), jnp.float32),
            pltpu.VMEM_SHARED((N_PAD, DIM), jnp.float32),
            pltpu.SemaphoreType.DMA,
            pltpu.SemaphoreType.DMA,
            pltpu.SemaphoreType.DMA,
            pltpu.SemaphoreType.DMA,
        ],
    )


def _batch_body(t0, t1, t2, p3a, p3b, idx, out_u, out_i, out_sq,
                idx_v, b0, b1, b2, b3, b4, sq_v, sem):
    cid = lax.axis_index("c")
    sid = lax.axis_index("s")
    wid = sid * NC + cid
    zf = jnp.zeros((16,), jnp.float32)
    for j in range(8):
        sq_v[pl.ds(j * 16, 16)] = zf
    nch = (NGCH - wid + (NW - 1)) // NW
    UCH = BATCH // ECHUNK

    def _chunk(ci, c):
        ch = wid + ci * NW
        base = ch * ECHUNK
        pltpu.sync_copy(idx.at[pl.ds(base, ECHUNK)], idx_v)
        d0 = pltpu.async_copy(t0.at[idx_v], b0, sem)
        d1 = pltpu.async_copy(t1.at[idx_v], b1, sem)
        d2 = pltpu.async_copy(t2.at[idx_v], b2, sem)
        d3 = pltpu.async_copy(p3a.at[idx_v], b3, sem)
        d4 = pltpu.async_copy(p3b.at[idx_v], b4, sem)
        d0.wait()
        d1.wait()
        d2.wait()
        d3.wait()
        d4.wait()

        def _row(i, cc):
            for j in range(8):
                s = pl.ds(j * 16, 16)
                x0 = b0[i, s]
                x1 = b1[i, s]
                x2 = b2[i, s]
                x3 = b3[i, s] + b4[i, s]
                b0[i, s] = (x0 + x1) + (x2 + x3)
                sq_v[s] = sq_v[s] + (x0 * x0 + x1 * x1) + (x2 * x2 + x3 * x3)
            return cc

        lax.fori_loop(0, ECHUNK, _row, 0)

        @pl.when(ch < UCH)
        def _():
            pltpu.sync_copy(b0, out_u.at[pl.ds(base, ECHUNK)])

        @pl.when(ch >= UCH)
        def _():
            pltpu.sync_copy(b0, out_i.at[pl.ds(base - BATCH, ECHUNK)])

        return c

    lax.fori_loop(0, nch, _chunk, 0)
    pltpu.sync_copy(sq_v, out_sq.at[pl.ds(wid * 128, 128)])


def _make_batch_gather():
    mesh = plsc.VectorSubcoreMesh(
        core_axis_name="c", subcore_axis_name="s", num_cores=NC, num_subcores=NS
    )
    return pl.kernel(
        _batch_body,
        out_type=(
            jax.ShapeDtypeStruct((BATCH, DIM), jnp.float32),
            jax.ShapeDtypeStruct((N_GROWS - BATCH, DIM), jnp.float32),
            jax.ShapeDtypeStruct((NW * 128,), jnp.float32),
        ),
        mesh=mesh,
        scratch_types=[
            pltpu.VMEM((ECHUNK,), jnp.int32),
            pltpu.VMEM((ECHUNK, DIM), jnp.float32),
            pltpu.VMEM((ECHUNK, DIM), jnp.float32),
            pltpu.VMEM((ECHUNK, DIM), jnp.float32),
            pltpu.VMEM((ECHUNK, DIM), jnp.float32),
            pltpu.VMEM((ECHUNK, DIM), jnp.float32),
            pltpu.VMEM((128,), jnp.float32),
            pltpu.SemaphoreType.DMA,
        ],
    )


def _comb_body(a_ref, b_ref, o_ref):
    o_ref[...] = a_ref[...] + b_ref[...]


def _combine(pa, pb):
    return pl.pallas_call(
        _comb_body,
        grid=(NZC,),
        in_specs=[pl.BlockSpec((ECHUNK, DIM), lambda i: (i, 0))] * 2,
        out_specs=pl.BlockSpec((ECHUNK, DIM), lambda i: (i, 0)),
        out_shape=jax.ShapeDtypeStruct((N_PAD, DIM), jnp.float32),
    )(pa, pb)


def _loss_body(u_ref, it_ref, sq_ref, o_ref):
    i = pl.program_id(0)
    u = u_ref[...]
    un = jnp.maximum(jnp.sqrt(jnp.sum(u * u, axis=1, keepdims=True)), 1e-12)
    uh = u / un
    it = it_ref[...]
    inorm = jnp.maximum(jnp.sqrt(jnp.sum(it * it, axis=2, keepdims=True)), 1e-12)
    y = jnp.sum((it / inorm) * uh[:, None, :], axis=2)
    logits = y / TEMPERATURE
    m = jnp.max(logits, axis=1, keepdims=True)
    lse = m + jnp.log(jnp.sum(jnp.exp(logits - m), axis=1, keepdims=True))
    part = jnp.sum(lse - logits[:, 0:1]) / BATCH

    @pl.when(i == 0)
    def _():
        o_ref[...] = jnp.full(
            (8, 128), DECAY * jnp.sum(sq_ref[...]) / (2.0 * BATCH), jnp.float32
        )

    o_ref[...] = o_ref[...] + part


def _loss(u_sum, items, sq):
    bb = 128
    return pl.pallas_call(
        _loss_body,
        grid=(BATCH // bb,),
        in_specs=[
            pl.BlockSpec((bb, DIM), lambda i: (i, 0)),
            pl.BlockSpec((bb, 1 + N_NEGS, DIM), lambda i: (i, 0, 0)),
            pl.BlockSpec((NW, 128), lambda i: (0, 0)),
        ],
        out_specs=pl.BlockSpec((8, 128), lambda i: (0, 0)),
        out_shape=jax.ShapeDtypeStruct((8, 128), jnp.float32),
    )(u_sum, items, sq)


def kernel(user_embed, item_embed, edge_index, edge_val, users, pos_items, neg_items):
    row = edge_index[0].astype(jnp.int32)
    col = edge_index[1].astype(jnp.int32)
    t0 = jnp.concatenate(
        [user_embed, item_embed,
         jnp.zeros((N_PAD - N_NODES, DIM), jnp.float32)], axis=0
    )

    npad = E_PAD - N_EDGES
    pad_idx = (jnp.arange(npad, dtype=jnp.int32) * 61) % N_NODES
    col2 = jnp.concatenate([col, pad_idx]).reshape(NCHP, ECHUNK)
    row2 = jnp.concatenate([row, pad_idx]).reshape(NCHP, ECHUNK)
    val2 = jnp.concatenate(
        [edge_val, jnp.zeros((npad,), jnp.float32)]
    ).reshape(NCHP, ECHUNK)

    hop = _make_hop()
    t1 = _combine(*hop(t0, col2, row2, val2))
    t2 = _combine(*hop(t1, col2, row2, val2))
    p3a, p3b = hop(t2, col2, row2, val2)

    item_idx = jnp.concatenate(
        [pos_items[:, None].astype(jnp.int32), neg_items.astype(jnp.int32)], axis=1
    )
    idx_all = jnp.concatenate(
        [users.astype(jnp.int32), (item_idx + N_USERS).reshape(-1)]
    )
    u_sum, item_sum, sq = _make_batch_gather()(t0, t1, t2, p3a, p3b, idx_all)
    items = item_sum.reshape(BATCH, 1 + N_NEGS, DIM)
    out = _loss(u_sum, items, sq.reshape(NW, 128))
    return out[0, 0]
